# baseline (speedup 1.0000x reference)
import jax
import jax.numpy as jnp
import numpy as np
from jax.experimental import pallas as pl
from jax.experimental.pallas import tpu as pltpu

_CHANNEL_NUM = 64
_REGIONS = [list(range(i * 8, (i + 1) * 8)) for i in range(8)]


def _channel_perm_build() -> tuple:
    perm = np.arange(_CHANNEL_NUM, dtype=np.int32)
    key = jax.random.key(42)
    for r, region in enumerate(_REGIONS):
        idx = np.array([c for c in region if c < _CHANNEL_NUM], dtype=np.int32)
        if len(idx) < 2:
            continue
        shuffled = jax.random.permutation(
            jax.random.fold_in(key, r), jnp.asarray(idx)
        )
        perm[idx] = np.asarray(shuffled)
    return tuple(int(x) for x in perm)


_PERM = _channel_perm_build()


def _channel_perm() -> tuple:
    return _PERM


def _dma_kernel(in_ref, out_ref, sem):
    copies = [
        pltpu.make_async_copy(
            in_ref.at[:, pc : pc + 1], out_ref.at[:, c : c + 1], sem
        )
        for c, pc in enumerate(_PERM)
    ]
    for cp in copies:
        cp.start()
    for cp in copies:
        cp.wait()


def kernel(data_tensor, domain_labels, aux_labels):
    del domain_labels, aux_labels
    b, c, one, t = data_tensor.shape
    out = pl.pallas_call(
        _dma_kernel,
        in_specs=[pl.BlockSpec(memory_space=pl.ANY)],
        out_specs=pl.BlockSpec(memory_space=pl.ANY),
        scratch_shapes=[pltpu.SemaphoreType.DMA],
        out_shape=jax.ShapeDtypeStruct((b, c, one, t), data_tensor.dtype),
    )(data_tensor)
    return out


# speedup vs baseline: 16.0712x; 16.0712x over previous
import jax
import jax.numpy as jnp
import numpy as np
from jax.experimental import pallas as pl
from jax.experimental.pallas import tpu as pltpu

_CHANNEL_NUM = 64
_REGIONS = [list(range(i * 8, (i + 1) * 8)) for i in range(8)]


_PERM = (
    1, 3, 5, 0, 2, 6, 7, 4,
    10, 8, 12, 13, 15, 14, 11, 9,
    21, 23, 16, 17, 20, 18, 22, 19,
    28, 29, 27, 26, 31, 30, 24, 25,
    38, 37, 33, 35, 34, 39, 36, 32,
    43, 41, 47, 42, 44, 40, 45, 46,
    49, 55, 54, 48, 53, 51, 52, 50,
    61, 56, 58, 63, 57, 59, 60, 62,
)


def _shuffle_kernel(in_ref, out_ref):
    for j in range(len(_PERM)):
        out_ref[:, j, :] = in_ref[:, _PERM[j], :]


_BLK_B = 4


def kernel(data_tensor, domain_labels, aux_labels):
    del domain_labels, aux_labels
    b, c, one, t = data_tensor.shape
    x = data_tensor.reshape(b, c, t)
    out = pl.pallas_call(
        _shuffle_kernel,
        grid=(b // _BLK_B,),
        in_specs=[pl.BlockSpec((_BLK_B, c, t), lambda bi: (bi, 0, 0))],
        out_specs=pl.BlockSpec((_BLK_B, c, t), lambda bi: (bi, 0, 0)),
        out_shape=jax.ShapeDtypeStruct((b, c, t), data_tensor.dtype),
    )(x)
    return out.reshape(b, c, one, t)
